# baseline (speedup 1.0000x reference)
import functools

import jax
import jax.numpy as jnp
from jax import lax
from jax.experimental import pallas as pl
from jax.experimental.pallas import tpu as pltpu
from jax.experimental.pallas import tpu_sc as plsc

B = 16384
D = 128
NUM_WORKERS = 32
ROWS_PER_WORKER = B // NUM_WORKERS
CHUNK = 128
NUM_CHUNKS = ROWS_PER_WORKER // CHUNK
LANES = 16
D_VECS = D // LANES


def _row_partial(urows, mrows, row):
    p0 = urows[row, pl.ds(0, LANES)] * mrows[row, pl.ds(0, LANES)]
    p1 = urows[row, pl.ds(LANES, LANES)] * mrows[row, pl.ds(LANES, LANES)]
    for j in range(2, D_VECS, 2):
        p0 = p0 + (urows[row, pl.ds(j * LANES, LANES)]
                   * mrows[row, pl.ds(j * LANES, LANES)])
        p1 = p1 + (urows[row, pl.ds((j + 1) * LANES, LANES)]
                   * mrows[row, pl.ds((j + 1) * LANES, LANES)])
    return p0 + p1


def _sc_kernel(uidx_hbm, midx_hbm, utab_hbm, mtab_hbm, out_hbm,
               uidx_v, midx_v, urows, mrows, outv, sems):
    wid = lax.axis_index("s") * 2 + lax.axis_index("c")
    pltpu.sync_copy(uidx_hbm.at[wid], uidx_v)
    pltpu.sync_copy(midx_hbm.at[wid], midx_v)
    iota = lax.iota(jnp.int32, LANES)
    last = jnp.full((LANES,), LANES - 1, jnp.int32)

    def issue(c):
        buf = c % 3
        cu = pltpu.async_copy(utab_hbm.at[uidx_v.at[c]], urows.at[buf],
                              sems.at[buf, 0])
        cm = pltpu.async_copy(mtab_hbm.at[midx_v.at[c]], mrows.at[buf],
                              sems.at[buf, 1])
        return cu, cm

    pending = [issue(0), issue(1)]
    for c in range(NUM_CHUNKS):
        cu, cm = pending.pop(0)
        if c + 2 < NUM_CHUNKS:
            pending.append(issue(c + 2))
        cu.wait()
        cm.wait()
        buf = c % 3
        ub = urows.at[buf]
        mb = mrows.at[buf]

        def group_body(g, _):
            row0 = g * LANES

            def row_body(r, accv):
                p = _row_partial(ub, mb, row0 + r)
                s = jnp.take_along_axis(plsc.cumsum(p), last, axis=0)
                return jnp.where(iota == r, s, accv)

            accv = lax.fori_loop(0, LANES, row_body,
                                 jnp.zeros((LANES,), jnp.float32),
                                 unroll=4)
            outv[pl.ds(c * CHUNK + row0, LANES)] = accv
            return 0

        lax.fori_loop(0, CHUNK // LANES, group_body, 0)

    base = wid * ROWS_PER_WORKER
    pltpu.sync_copy(outv, out_hbm.at[pl.ds(base, ROWS_PER_WORKER)])


@jax.jit
def _run(uidx, midx, user_table, movie_table):
    mesh = plsc.VectorSubcoreMesh(core_axis_name="c", subcore_axis_name="s")
    fn = functools.partial(
        pl.kernel,
        mesh=mesh,
        compiler_params=pltpu.CompilerParams(needs_layout_passes=False),
        out_type=jax.ShapeDtypeStruct((B,), jnp.float32),
        scratch_types=[
            pltpu.VMEM((NUM_CHUNKS, CHUNK), jnp.int32),
            pltpu.VMEM((NUM_CHUNKS, CHUNK), jnp.int32),
            pltpu.VMEM((3, CHUNK, D), jnp.float32),
            pltpu.VMEM((3, CHUNK, D), jnp.float32),
            pltpu.VMEM((ROWS_PER_WORKER,), jnp.float32),
            pltpu.SemaphoreType.DMA((3, 2)),
        ],
    )(_sc_kernel)
    return fn(uidx, midx, user_table, movie_table)


def kernel(inputs, user_table, movie_table):
    idx = inputs.astype(jnp.int32)
    uidx = idx[:, 0].reshape(NUM_WORKERS, NUM_CHUNKS, CHUNK)
    midx = idx[:, 1].reshape(NUM_WORKERS, NUM_CHUNKS, CHUNK)
    out = _run(uidx, midx, user_table, movie_table)
    return out.reshape(B, 1)


# speedup vs baseline: 1.0354x; 1.0354x over previous
import functools

import jax
import jax.numpy as jnp
from jax import lax
from jax.experimental import pallas as pl
from jax.experimental.pallas import tpu as pltpu
from jax.experimental.pallas import tpu_sc as plsc

B = 16384
D = 128
NUM_WORKERS = 32
ROWS_PER_WORKER = B // NUM_WORKERS
MAX_CHUNK = 128
LANES = 16
D_VECS = D // LANES

_CHUNKS = ((0, 32), (32, 96), (128, 128), (256, 128), (384, 128))
_NBUF = 3


def _row_partial(urows, mrows, row):
    p0 = urows[row, pl.ds(0, LANES)] * mrows[row, pl.ds(0, LANES)]
    p1 = urows[row, pl.ds(LANES, LANES)] * mrows[row, pl.ds(LANES, LANES)]
    for j in range(2, D_VECS, 2):
        p0 = p0 + (urows[row, pl.ds(j * LANES, LANES)]
                   * mrows[row, pl.ds(j * LANES, LANES)])
        p1 = p1 + (urows[row, pl.ds((j + 1) * LANES, LANES)]
                   * mrows[row, pl.ds((j + 1) * LANES, LANES)])
    return p0 + p1


def _sc_kernel(uidx_hbm, midx_hbm, utab_hbm, mtab_hbm, out_hbm,
               uidx_v, midx_v, urows, mrows, outv, sems):
    wid = lax.axis_index("s") * 2 + lax.axis_index("c")
    pltpu.sync_copy(uidx_hbm.at[wid], uidx_v)
    pltpu.sync_copy(midx_hbm.at[wid], midx_v)
    iota = lax.iota(jnp.int32, LANES)

    def issue(k):
        off, n = _CHUNKS[k]
        buf = k % _NBUF
        cu = pltpu.async_copy(utab_hbm.at[uidx_v.at[pl.ds(off, n)]],
                              urows.at[buf, pl.ds(0, n)], sems.at[buf, 0])
        cm = pltpu.async_copy(mtab_hbm.at[midx_v.at[pl.ds(off, n)]],
                              mrows.at[buf, pl.ds(0, n)], sems.at[buf, 1])
        return cu, cm

    pending = [issue(0), issue(1)]
    for k in range(len(_CHUNKS)):
        cu, cm = pending.pop(0)
        if k + 2 < len(_CHUNKS):
            pending.append(issue(k + 2))
        cu.wait()
        cm.wait()
        off, n = _CHUNKS[k]
        buf = k % _NBUF
        ub = urows.at[buf]
        mb = mrows.at[buf]

        def group_body(g, _):
            row0 = g * LANES

            def row_body(r, accv):
                p = _row_partial(ub, mb, row0 + r)
                return jnp.where(iota == r, jnp.sum(p), accv)

            accv = lax.fori_loop(0, LANES, row_body,
                                 jnp.zeros((LANES,), jnp.float32),
                                 unroll=4)
            outv[pl.ds(off + row0, LANES)] = accv
            return 0

        lax.fori_loop(0, n // LANES, group_body, 0)

    base = wid * ROWS_PER_WORKER
    pltpu.sync_copy(outv, out_hbm.at[pl.ds(base, ROWS_PER_WORKER)])


@jax.jit
def _run(uidx, midx, user_table, movie_table):
    mesh = plsc.VectorSubcoreMesh(core_axis_name="c", subcore_axis_name="s")
    fn = functools.partial(
        pl.kernel,
        mesh=mesh,
        compiler_params=pltpu.CompilerParams(needs_layout_passes=False),
        out_type=jax.ShapeDtypeStruct((B,), jnp.float32),
        scratch_types=[
            pltpu.VMEM((ROWS_PER_WORKER,), jnp.int32),
            pltpu.VMEM((ROWS_PER_WORKER,), jnp.int32),
            pltpu.VMEM((_NBUF, MAX_CHUNK, D), jnp.float32),
            pltpu.VMEM((_NBUF, MAX_CHUNK, D), jnp.float32),
            pltpu.VMEM((ROWS_PER_WORKER,), jnp.float32),
            pltpu.SemaphoreType.DMA((_NBUF, 2)),
        ],
    )(_sc_kernel)
    return fn(uidx, midx, user_table, movie_table)


def kernel(inputs, user_table, movie_table):
    idx = inputs.astype(jnp.int32)
    uidx = idx[:, 0].reshape(NUM_WORKERS, ROWS_PER_WORKER)
    midx = idx[:, 1].reshape(NUM_WORKERS, ROWS_PER_WORKER)
    out = _run(uidx, midx, user_table, movie_table)
    return out.reshape(B, 1)
